# baseline (speedup 1.0000x reference)
import functools
import math

import jax
import jax.numpy as jnp
from jax import lax
from jax.experimental import pallas as pl
from jax.experimental.pallas import tpu as pltpu
from jax.experimental.pallas import tpu_sc as plsc

N = 50000
NP = 51200
NF = NP // 8
E = 800000
EP = 819200
K1 = 45000
K2 = 40500
RPT = NP // 16
EPT = EP // 32
NCH = RPT // 128
ECH = EPT // 128

f32 = jnp.float32
i32 = jnp.int32


def _prep_body(atc_ref, ord_ref, cate_ref, wti_ref, wtc_ref, prow_ref, ptyp_ref,
               atcT_ref, ordT_ref, cateT_ref, promoT_ref):
    i = pl.program_id(0)
    atcT_ref[...] = jnp.dot(atc_ref[...], wti_ref[...], preferred_element_type=f32)
    ordT_ref[...] = jnp.dot(ord_ref[...], wti_ref[...], preferred_element_type=f32)

    @pl.when(i == 0)
    def _():
        cateT_ref[...] = jnp.dot(cate_ref[...], wtc_ref[...], preferred_element_type=f32)
        promoT_ref[...] = (
            jnp.dot(prow_ref[...], wti_ref[...], preferred_element_type=f32)
            + jnp.dot(ptyp_ref[...], wtc_ref[...], preferred_element_type=f32))


def _prep_call(atc, ordt, cate, wti, wtc, prow4, ptyp4):
    nb = 50
    blk = 100000 // nb
    return pl.pallas_call(
        _prep_body,
        grid=(nb,),
        in_specs=[
            pl.BlockSpec((blk, 64), lambda i: (i, 0)),
            pl.BlockSpec((blk, 64), lambda i: (i, 0)),
            pl.BlockSpec((1000, 16), lambda i: (0, 0)),
            pl.BlockSpec((64, 32), lambda i: (0, 0)),
            pl.BlockSpec((16, 32), lambda i: (0, 0)),
            pl.BlockSpec((4, 64), lambda i: (0, 0)),
            pl.BlockSpec((4, 16), lambda i: (0, 0)),
        ],
        out_specs=[
            pl.BlockSpec((blk, 32), lambda i: (i, 0)),
            pl.BlockSpec((blk, 32), lambda i: (i, 0)),
            pl.BlockSpec((1000, 32), lambda i: (0, 0)),
            pl.BlockSpec((4, 32), lambda i: (0, 0)),
        ],
        out_shape=[
            jax.ShapeDtypeStruct((100000, 32), f32),
            jax.ShapeDtypeStruct((100000, 32), f32),
            jax.ShapeDtypeStruct((1000, 32), f32),
            jax.ShapeDtypeStruct((4, 32), f32),
        ],
    )(atc, ordt, cate, wti, wtc, prow4, ptyp4)


_MESH = dict(core_axis_name="c", subcore_axis_name="s")


def _zero16(ref, nrow):
    def b(r, _):
        ref[r, :] = jnp.zeros((16,), f32)
        return _
    lax.fori_loop(0, nrow, b, None)


def _edge_pass(g, P, S, CNT, cbuf, srcs, dsts, sv2d, dv2d, Grow, sem, wid,
               do_counts):
    pltpu.sync_copy(srcs.at[g, pl.ds(wid * ECH, ECH)], sv2d)
    pltpu.sync_copy(dsts.at[g, pl.ds(wid * ECH, ECH)], dv2d)

    def chunk(j, _):
        pltpu.async_copy(P.at[sv2d.at[j]], Grow, sem).wait()
        pltpu.sync_copy(Grow, S.at[dv2d.at[j]], add=True)
        if do_counts:
            def crow(i, _):
                s16 = sv2d[j, pl.ds(i * 16, 16)]
                d16 = dv2d[j, pl.ds(i * 16, 16)]
                cbuf[pl.ds(i * 16, 16)] = jnp.where(s16 == d16, 1.0, 0.0).astype(f32)
                return _
            lax.fori_loop(0, 8, crow, None)
            pltpu.sync_copy(cbuf, CNT.at[dv2d.at[j]], add=True)
        return _

    lax.fori_loop(0, ECH, chunk, None)


GRPT = NP // 32
GCH = GRPT // 64


def _sc_gather_body(atcT, ordT, cateT, item_ids, cate_ids, promoT,
                    xn_o, xr_o,
                    it1d, ct1d, Gi2, Gc2, bufN2, bufR2, ptT, sem, semW):
    cid = lax.axis_index("c")
    sid = lax.axis_index("s")
    wid = cid * 16 + sid
    rowbase = wid * GRPT
    pltpu.sync_copy(promoT, ptT)

    for g in range(4):
        tbl = atcT if g in (0, 2) else ordT
        pltpu.sync_copy(item_ids.at[pl.ds(g * NP + rowbase, GRPT)], it1d)
        pltpu.sync_copy(cate_ids.at[pl.ds(g * NP + rowbase, GRPT)], ct1d)

        def fire_gathers(c, slot):
            pltpu.async_copy(tbl.at[it1d.at[pl.ds(c * 64, 64)]],
                             Gi2.at[slot], sem)
            pltpu.async_copy(cateT.at[ct1d.at[pl.ds(c * 64, 64)]],
                             Gc2.at[slot], sem)

        def drain_gathers(c, slot):
            pltpu.make_async_copy(tbl.at[it1d.at[pl.ds(c * 64, 64)]],
                                  Gi2.at[slot], sem).wait()
            pltpu.make_async_copy(cateT.at[ct1d.at[pl.ds(c * 64, 64)]],
                                  Gc2.at[slot], sem).wait()

        def fire_writes(c, slot):
            base = rowbase + c * 64
            pltpu.async_copy(bufN2.at[slot], xn_o.at[g, pl.ds(base, 64)], semW)
            pltpu.async_copy(bufR2.at[slot], xr_o.at[g, pl.ds(base, 64)], semW)

        def drain_writes(c, slot):
            base = rowbase + c * 64
            pltpu.make_async_copy(bufN2.at[slot], xn_o.at[g, pl.ds(base, 64)],
                                  semW).wait()
            pltpu.make_async_copy(bufR2.at[slot], xr_o.at[g, pl.ds(base, 64)],
                                  semW).wait()

        fire_gathers(0, 0)

        def cbody(c, _):
            slot = lax.rem(c, 2)

            @pl.when(c >= 2)
            def _():
                drain_writes(c - 2, slot)
            drain_gathers(c, slot)

            @pl.when(c + 1 < GCH)
            def _():
                fire_gathers(c + 1, 1 - slot)

            def addrow(r, _):
                bufN2[slot, r, :] = Gi2[slot, r, 0:16] + Gc2[slot, r, 0:16]
                bufR2[slot, r, :] = (Gi2[slot, r, pl.ds(16, 16)]
                                     + Gc2[slot, r, pl.ds(16, 16)])
                return _
            lax.fori_loop(0, 64, addrow, None)

            @pl.when((wid == 0) & (c == 0))
            def _():
                bufN2[0, 0, :] = ptT[g, 0:16]
                bufR2[0, 0, :] = ptT[g, pl.ds(16, 16)]

            fire_writes(c, slot)
            return _

        lax.fori_loop(0, GCH, cbody, None)
        drain_writes(GCH - 2, GCH % 2)
        drain_writes(GCH - 1, (GCH - 1) % 2)


def _sc_gather_call(atcT, ordT, cateT, item1d, cate1d, promoT):
    kern = pl.kernel(
        _sc_gather_body,
        out_type=[
            jax.ShapeDtypeStruct((4, NP, 16), f32),
            jax.ShapeDtypeStruct((4, NP, 16), f32),
        ],
        mesh=plsc.VectorSubcoreMesh(**_MESH),
        compiler_params=pltpu.CompilerParams(use_tc_tiling_on_sc=False),
        scratch_types=[
            pltpu.VMEM((GRPT,), i32),
            pltpu.VMEM((GRPT,), i32),
            pltpu.VMEM((2, 64, 32), f32),
            pltpu.VMEM((2, 64, 32), f32),
            pltpu.VMEM((2, 64, 16), f32),
            pltpu.VMEM((2, 64, 16), f32),
            pltpu.VMEM((4, 32), f32),
            pltpu.SemaphoreType.DMA,
            pltpu.SemaphoreType.DMA,
        ],
    )
    return kern(atcT, ordT, cateT, item1d, cate1d, promoT)


def _make_sc_edge_body(do_counts):
    G = 8
    NGRP = ECH // G

    def body(*args):
        if do_counts:
            (tblH, srcs, dsts, Sp_o, cnt_o,
             bufN, sv4, dv4, Grow2, cbuf2, zb, zc, S, CNT,
             sem, semS, semI) = args
        else:
            (tblH, srcs, dsts, Sp_o,
             bufN, sv4, dv4, Grow2, cbuf2, zb, zc, S, CNT,
             sem, semS, semI) = args
            cnt_o = None
        cid = lax.axis_index("c")
        sid = lax.axis_index("s")
        wid = cid * 16 + sid
        rowbase = sid * RPT
        _zero16(zb, 128)

        def zc_b(i, _):
            zc[pl.ds(i * 16, 16)] = jnp.zeros((16,), f32)
            return _
        lax.fori_loop(0, 8, zc_b, None)

        GB = G * 128

        for g in range(4):
            for c in range(NCH):
                pltpu.async_copy(zb, S.at[pl.ds(rowbase + c * 128, 128)], sem)
                if do_counts:
                    pltpu.async_copy(zc, CNT.at[pl.ds(rowbase + c * 128, 128)], sem)
            for c in range(NCH):
                pltpu.make_async_copy(zb, S.at[pl.ds(rowbase + c * 128, 128)], sem).wait()
                if do_counts:
                    pltpu.make_async_copy(zc, CNT.at[pl.ds(rowbase + c * 128, 128)], sem).wait()
            plsc.subcore_barrier()

            ebase = wid * ECH * 128
            tblg = tblH.at[g]
            srcf = srcs.at[g]
            dstf = dsts.at[g]

            def fire_idx(grp):
                s4 = lax.rem(grp, 4)
                row0 = wid * ECH + grp * G
                pltpu.async_copy(srcf.at[pl.ds(row0, G)],
                                 sv4.at[s4], semI)
                pltpu.async_copy(dstf.at[pl.ds(row0, G)],
                                 dv4.at[s4], semI)

            def drain_idx(grp):
                s4 = lax.rem(grp, 4)
                row0 = wid * ECH + grp * G
                pltpu.make_async_copy(srcf.at[pl.ds(row0, G)],
                                      sv4.at[s4], semI).wait()
                pltpu.make_async_copy(dstf.at[pl.ds(row0, G)],
                                      dv4.at[s4], semI).wait()

            def fire_gathers(grp, s2):
                s4 = lax.rem(grp, 4)
                for k in range(G):
                    pltpu.async_copy(
                        tblg.at[sv4.at[s4, k]],
                        Grow2.at[s2, pl.ds(k * 128, 128)], sem)

            def drain_gathers(grp, s2):
                s4 = lax.rem(grp, 4)
                for k in range(G):
                    pltpu.make_async_copy(
                        tblg.at[sv4.at[s4, k]],
                        Grow2.at[s2, pl.ds(k * 128, 128)], sem).wait()

            def fire_scatters(grp, s2):
                s4 = lax.rem(grp, 4)
                for k in range(G):
                    pltpu.async_copy(
                        Grow2.at[s2, pl.ds(k * 128, 128)],
                        S.at[dv4.at[s4, k]], semS, add=True)
                    if do_counts:
                        def crow(i, _):
                            s16 = sv4[s4, k, pl.ds(i * 16, 16)]
                            d16 = dv4[s4, k, pl.ds(i * 16, 16)]
                            cbuf2[s2, pl.ds(k * 128 + i * 16, 16)] = jnp.where(
                                s16 == d16, 1.0, 0.0).astype(f32)
                            return _
                        lax.fori_loop(0, 8, crow, None)
                        pltpu.async_copy(
                            cbuf2.at[s2, pl.ds(k * 128, 128)],
                            CNT.at[dv4.at[s4, k]], semS, add=True)

            def drain_scatters(grp, s2):
                s4 = lax.rem(grp, 4)
                for k in range(G):
                    pltpu.make_async_copy(
                        Grow2.at[s2, pl.ds(k * 128, 128)],
                        S.at[dv4.at[s4, k]], semS).wait()
                    if do_counts:
                        pltpu.make_async_copy(
                            cbuf2.at[s2, pl.ds(k * 128, 128)],
                            CNT.at[dv4.at[s4, k]], semS).wait()

            fire_idx(0)
            fire_idx(1)
            drain_idx(0)
            fire_gathers(0, 0)

            def grp_body(grp, _):
                s3 = lax.rem(grp, 3)
                drain_gathers(grp, s3)
                fire_scatters(grp, s3)

                @pl.when(grp >= 2)
                def _():
                    drain_scatters(grp - 2, lax.rem(grp + 1, 3))

                @pl.when(grp + 2 < NGRP)
                def _():
                    fire_idx(grp + 2)

                @pl.when(grp + 1 < NGRP)
                def _():
                    drain_idx(grp + 1)
                    fire_gathers(grp + 1, lax.rem(grp + 1, 3))
                return _

            lax.fori_loop(0, NGRP, grp_body, None)
            drain_scatters(NGRP - 2, (NGRP - 2) % 3)
            drain_scatters(NGRP - 1, (NGRP - 1) % 3)
            plsc.subcore_barrier()

            def ochunk(c, _):
                base = rowbase + c * 128
                pltpu.sync_copy(S.at[pl.ds(base, 128)], bufN)
                pltpu.sync_copy(bufN, Sp_o.at[cid, g, pl.ds(base, 128)])
                if do_counts:
                    pltpu.sync_copy(CNT.at[pl.ds(base, 128)], cbuf2.at[0, pl.ds(0, 128)])
                    pltpu.sync_copy(
                        cbuf2.at[0, pl.ds(0, 128)],
                        cnt_o.at[pl.ds((cid * 4 + g) * NP + base, 128)])
                return _
            lax.fori_loop(0, NCH, ochunk, None)
    return body


def _sc_edge_call(tbl, srcs3, dsts3, do_counts):
    out_type = [jax.ShapeDtypeStruct((2, 4, NP, 16), f32)]
    if do_counts:
        out_type.append(jax.ShapeDtypeStruct((2 * 4 * NP,), f32))
    kern = pl.kernel(
        _make_sc_edge_body(do_counts),
        out_type=out_type,
        mesh=plsc.VectorSubcoreMesh(**_MESH),
        compiler_params=pltpu.CompilerParams(use_tc_tiling_on_sc=False),
        scratch_types=[
            pltpu.VMEM((128, 16), f32),
            pltpu.VMEM((4, 8, 128), i32),
            pltpu.VMEM((4, 8, 128), i32),
            pltpu.VMEM((3, 1024, 16), f32),
            pltpu.VMEM((3, 1024), f32),
            pltpu.VMEM((128, 16), f32),
            pltpu.VMEM((128,), f32),
            pltpu.VMEM_SHARED((NP, 16), f32),
            pltpu.VMEM_SHARED((NP,), f32),
            pltpu.SemaphoreType.DMA,
            pltpu.SemaphoreType.DMA,
            pltpu.SemaphoreType.DMA,
        ],
    )
    return kern(tbl, srcs3, dsts3)


def _fkey(x):
    k = lax.bitcast_convert_type(x, i32)
    return k ^ jnp.where(k < 0, jnp.int32(0x7FFFFFFF), jnp.int32(0))


def _kth_threshold(keyT, k):
    def b(i, T):
        cand = T + (jnp.int32(1) << (30 - i))
        cnt = jnp.sum((keyT >= cand).astype(i32))
        return jnp.where(cnt >= k, cand, T)
    return lax.fori_loop(0, 31, b, jnp.int32(-2**31))


def _fold8(v, op):
    parts = [lax.slice_in_dim(v, k * 16, (k + 1) * 16) for k in range(8)]
    return functools.reduce(op, parts)


def _select_gate_pool(h, score_raw, alive, kcount, E8):
    scorem = jnp.where(alive, score_raw, -jnp.inf)
    key = _fkey(scorem)
    keyT = _fkey(jnp.transpose(scorem))
    T = _kth_threshold(keyT, kcount)
    keep = key >= T
    keepf = keep.astype(f32)
    gate = jnp.tanh(jnp.where(alive, score_raw, 0.0)) * keepf
    gexp = jnp.dot(gate, E8, preferred_element_type=f32)
    kexp = jnp.dot(keepf, E8, preferred_element_type=f32)
    g = h * gexp
    colmax = jnp.max(jnp.where(kexp > 0.0, g, -jnp.inf), axis=0)
    colsum = jnp.sum(g, axis=0)
    m16 = _fold8(colmax, jnp.maximum)
    s16 = _fold8(colsum, jnp.add) * (1.0 / kcount)
    return g, keepf, jnp.concatenate([m16, s16])


NSL = 4
NFS = NF // NSL


def _tc_h1_body(xnf_ref, xrf_ref, S0p_ref, cnt_ref, E8_ref, b0t_ref, h1_ref):
    S0 = S0p_ref[0, 0] + S0p_ref[1, 0]
    c8 = cnt_ref[0, 0] + cnt_ref[1, 0]
    cexp = jnp.dot(c8, E8_ref[...], preferred_element_type=f32)
    h1_ref[0] = jnp.maximum(
        S0 + (1.0 - cexp) * xnf_ref[0] + xrf_ref[0] + b0t_ref[...], 0.0)


def _tc_h1_call(xnf, xrf, S0pf, cnt8, E8, b0t):
    return pl.pallas_call(
        _tc_h1_body,
        grid=(4, NSL),
        in_specs=[
            pl.BlockSpec((1, NFS, 128), lambda i, j: (i, j, 0)),
            pl.BlockSpec((1, NFS, 128), lambda i, j: (i, j, 0)),
            pl.BlockSpec((2, 1, NFS, 128), lambda i, j: (0, i, j, 0)),
            pl.BlockSpec((2, 1, NFS, 8), lambda i, j: (0, i, j, 0)),
            pl.BlockSpec((8, 128), lambda i, j: (0, 0)),
            pl.BlockSpec((1, 128), lambda i, j: (0, 0)),
        ],
        out_specs=[pl.BlockSpec((1, NFS, 128), lambda i, j: (i, j, 0))],
        out_shape=[jax.ShapeDtypeStruct((4, NF, 128), f32)],
    )(xnf, xrf, S0pf, cnt8, E8, b0t)


def _tc_b_body(h1_ref, E8_ref, P0n_ref, Wn1b_ref, Wr1b_ref,
               pre2_ref, root2_ref, keep_ref, pool1_ref):
    h1 = h1_ref[0]
    E8 = E8_ref[...]
    score = jnp.dot(h1, P0n_ref[...], preferred_element_type=f32)
    r_iota = lax.broadcasted_iota(i32, (NF, 8), 0)
    k_iota = lax.broadcasted_iota(i32, (NF, 8), 1)
    valid = (r_iota * 8 + k_iota) < N
    g1, keepf, pool = _select_gate_pool(h1, score, valid, K1, E8)
    pool1_ref[0, 0] = pool
    keep_ref[0] = keepf
    pre2_ref[0] = jnp.dot(g1, Wn1b_ref[...], preferred_element_type=f32)
    root2_ref[0] = jnp.dot(g1, Wr1b_ref[...], preferred_element_type=f32)


def _tc_b_call(h1f, E8, P0n, Wn1b, Wr1b):
    return pl.pallas_call(
        _tc_b_body,
        grid=(4,),
        in_specs=[
            pl.BlockSpec((1, NF, 128), lambda i: (i, 0, 0)),
            pl.BlockSpec((8, 128), lambda i: (0, 0)),
            pl.BlockSpec((128, 8), lambda i: (0, 0)),
            pl.BlockSpec((128, 128), lambda i: (0, 0)),
            pl.BlockSpec((128, 128), lambda i: (0, 0)),
        ],
        out_specs=[
            pl.BlockSpec((1, NF, 128), lambda i: (i, 0, 0)),
            pl.BlockSpec((1, NF, 128), lambda i: (i, 0, 0)),
            pl.BlockSpec((1, NF, 8), lambda i: (i, 0, 0)),
            pl.BlockSpec((1, 1, 32), lambda i: (i, 0, 0)),
        ],
        out_shape=[
            jax.ShapeDtypeStruct((4, NF, 128), f32),
            jax.ShapeDtypeStruct((4, NF, 128), f32),
            jax.ShapeDtypeStruct((4, NF, 8), f32),
            jax.ShapeDtypeStruct((4, 1, 32), f32),
        ],
    )(h1f, E8, P0n, Wn1b, Wr1b)


def _tc_h2_body(S1p_ref, pre2_ref, root2_ref, keep_ref, cnt_ref, E8_ref,
                b1t_ref, h2_ref):
    S1 = S1p_ref[0, 0] + S1p_ref[1, 0]
    c8 = cnt_ref[0, 0] + cnt_ref[1, 0]
    E8 = E8_ref[...]
    cexp = jnp.dot(c8, E8, preferred_element_type=f32)
    kexp = jnp.dot(keep_ref[0], E8, preferred_element_type=f32)
    h2_ref[0] = kexp * jnp.maximum(
        S1 + (1.0 - cexp) * pre2_ref[0] + root2_ref[0] + b1t_ref[...], 0.0)


def _tc_h2_call(S1pf, pre2f, root2f, keep8, cnt8, E8, b1t):
    return pl.pallas_call(
        _tc_h2_body,
        grid=(4, NSL),
        in_specs=[
            pl.BlockSpec((2, 1, NFS, 128), lambda i, j: (0, i, j, 0)),
            pl.BlockSpec((1, NFS, 128), lambda i, j: (i, j, 0)),
            pl.BlockSpec((1, NFS, 128), lambda i, j: (i, j, 0)),
            pl.BlockSpec((1, NFS, 8), lambda i, j: (i, j, 0)),
            pl.BlockSpec((2, 1, NFS, 8), lambda i, j: (0, i, j, 0)),
            pl.BlockSpec((8, 128), lambda i, j: (0, 0)),
            pl.BlockSpec((1, 128), lambda i, j: (0, 0)),
        ],
        out_specs=[pl.BlockSpec((1, NFS, 128), lambda i, j: (i, j, 0))],
        out_shape=[jax.ShapeDtypeStruct((4, NF, 128), f32)],
    )(S1pf, pre2f, root2f, keep8, cnt8, E8, b1t)


def _tc_d1_body(h2_ref, keep_ref, pool1_ref, E8_ref, P1n_ref, pooltot_ref):
    h2 = h2_ref[0]
    keep1 = keep_ref[0]
    E8 = E8_ref[...]
    score = jnp.dot(h2, P1n_ref[...], preferred_element_type=f32)
    alive = keep1 > 0.0
    _, _, pool = _select_gate_pool(h2, score, alive, K2, E8)
    pooltot_ref[0, 0] = pool1_ref[0, 0] + pool


def _tc_d1_call(h2f, keep8, pool1, E8, P1n):
    return pl.pallas_call(
        _tc_d1_body,
        grid=(4,),
        in_specs=[
            pl.BlockSpec((1, NF, 128), lambda i: (i, 0, 0)),
            pl.BlockSpec((1, NF, 8), lambda i: (i, 0, 0)),
            pl.BlockSpec((1, 1, 32), lambda i: (i, 0, 0)),
            pl.BlockSpec((8, 128), lambda i: (0, 0)),
            pl.BlockSpec((128, 8), lambda i: (0, 0)),
        ],
        out_specs=[pl.BlockSpec((1, 1, 32), lambda i: (i, 0, 0))],
        out_shape=[jax.ShapeDtypeStruct((4, 1, 32), f32)],
    )(h2f, keep8, pool1, E8, P1n)


def _tc_d2_body(pool_ref, wl_ref, bl_ref, prow_ref, sig_ref, fc_ref):
    xx = jnp.concatenate(
        [pool_ref[0], pool_ref[1], pool_ref[2], pool_ref[3]],
        axis=1)
    fc = jnp.dot(xx, wl_ref[...], preferred_element_type=f32) + bl_ref[...]
    pv = prow_ref[0:1, :]
    s = jnp.sum(pv * fc)
    sig_ref[...] = (1.0 / (1.0 + jnp.exp(-s)))[None]
    fc_ref[...] = fc[0]


def _tc_d2_call(pooltot, W_lin, b_lin, prows):
    return pl.pallas_call(
        _tc_d2_body,
        out_shape=[
            jax.ShapeDtypeStruct((1,), f32),
            jax.ShapeDtypeStruct((64,), f32),
        ],
    )(pooltot, W_lin, b_lin, prows)


def kernel(x_b_atc, edge_index_b_atc, x_b_ord, edge_index_b_ord,
           x_a_atc, edge_index_a_atc, x_a_ord, edge_index_a_ord,
           atc_emb, ord_emb, promo_emb, promo_type_emb, cate_emb,
           Wr0, Wn0, b0, Wr1, Wn1, b1, p0, p1, W_lin, b_lin):
    xs = [x_b_atc, x_b_ord, x_a_atc, x_a_ord]
    eis = [edge_index_b_atc, edge_index_b_ord, edge_index_a_atc,
           edge_index_a_ord]

    npad = NP - N
    item3 = jnp.stack([
        jnp.pad(x[:, 0] % 100000, (0, npad)) for x in xs]).reshape(4 * NP)
    cate3 = jnp.stack([
        jnp.pad(x[:, 1] % 1000, (0, npad)) for x in xs]).reshape(4 * NP)
    epad = EP - E
    srcs3 = jnp.stack([
        jnp.pad(ei[0], (0, epad)) for ei in eis]).reshape(4, EP // 128, 128)
    padvals = N + (jnp.arange(epad, dtype=jnp.int32) % (NP - N))
    dsts3 = jnp.stack([
        jnp.concatenate([ei[1], padvals]) for ei in eis
    ]).reshape(4, EP // 128, 128)
    pids4 = jnp.stack([x[0, 0] % 100000 for x in xs])
    ptypes4 = jnp.stack([x[0, 1] % 100 for x in xs])
    prow4 = jnp.take(promo_emb, pids4, axis=0)
    ptyp4 = jnp.take(promo_type_emb, ptypes4, axis=0)

    wti = jnp.concatenate([Wn0[:64], Wr0[:64]], axis=1)
    wtc = jnp.concatenate([Wn0[64:], Wr0[64:]], axis=1)
    eye8 = jnp.eye(8, dtype=f32)
    E8 = jnp.kron(eye8, jnp.ones((1, 16), f32))
    P0n = jnp.kron(eye8, (p0 / jnp.linalg.norm(p0))[:, None])
    P1n = jnp.kron(eye8, (p1 / jnp.linalg.norm(p1))[:, None])
    Wn1b = jnp.kron(eye8, Wn1)
    Wr1b = jnp.kron(eye8, Wr1)
    b0t = jnp.tile(b0, 8)[None, :]
    b1t = jnp.tile(b1, 8)[None, :]

    atcT, ordT, cateT, promoT = _prep_call(
        atc_emb, ord_emb, cate_emb, wti, wtc, prow4, ptyp4)

    xn, xr = _sc_gather_call(atcT, ordT, cateT, item3, cate3, promoT)
    S0p, cnt = _sc_edge_call(xn, srcs3, dsts3, do_counts=True)

    xnf = xn.reshape(4, NF, 128)
    xrf = xr.reshape(4, NF, 128)
    S0pf = S0p.reshape(2, 4, NF, 128)
    cnt8 = cnt.reshape(2, 4, NF, 8)

    h1f = _tc_h1_call(xnf, xrf, S0pf, cnt8, E8, b0t)[0]
    pre2f, root2f, keep8, pool1 = _tc_b_call(h1f, E8, P0n, Wn1b, Wr1b)

    S1p = _sc_edge_call(pre2f.reshape(4, NP, 16), srcs3, dsts3,
                        do_counts=False)[0]
    S1pf = S1p.reshape(2, 4, NF, 128)

    h2f = _tc_h2_call(S1pf, pre2f, root2f, keep8, cnt8, E8, b1t)[0]
    pooltot = _tc_d1_call(h2f, keep8, pool1, E8, P1n)[0]

    sig, fc = _tc_d2_call(pooltot, W_lin, b_lin, prow4)

    return (sig, xs[3][0, 0], prow4[0], xs[3][0, 1], fc)


# speedup vs baseline: 1.6353x; 1.6353x over previous
import functools
import math

import jax
import jax.numpy as jnp
from jax import lax
from jax.experimental import pallas as pl
from jax.experimental.pallas import tpu as pltpu
from jax.experimental.pallas import tpu_sc as plsc

N = 50000
NP = 51200
NF = NP // 8
E = 800000
EP = 819200
K1 = 45000
K2 = 40500
RPT = NP // 16
EPT = EP // 32
NCH = RPT // 128
ECH = EPT // 128

f32 = jnp.float32
i32 = jnp.int32


def _prep_body(atc_ref, ord_ref, cate_ref, wti_ref, wtc_ref, prow_ref, ptyp_ref,
               atcT_ref, ordT_ref, cateT_ref, promoT_ref):
    i = pl.program_id(0)
    atcT_ref[...] = jnp.dot(atc_ref[...], wti_ref[...], preferred_element_type=f32)
    ordT_ref[...] = jnp.dot(ord_ref[...], wti_ref[...], preferred_element_type=f32)

    @pl.when(i == 0)
    def _():
        cateT_ref[...] = jnp.dot(cate_ref[...], wtc_ref[...], preferred_element_type=f32)
        promoT_ref[...] = (
            jnp.dot(prow_ref[...], wti_ref[...], preferred_element_type=f32)
            + jnp.dot(ptyp_ref[...], wtc_ref[...], preferred_element_type=f32))


def _prep_call(atc, ordt, cate, wti, wtc, prow4, ptyp4):
    nb = 50
    blk = 100000 // nb
    return pl.pallas_call(
        _prep_body,
        grid=(nb,),
        in_specs=[
            pl.BlockSpec((blk, 64), lambda i: (i, 0)),
            pl.BlockSpec((blk, 64), lambda i: (i, 0)),
            pl.BlockSpec((1000, 16), lambda i: (0, 0)),
            pl.BlockSpec((64, 32), lambda i: (0, 0)),
            pl.BlockSpec((16, 32), lambda i: (0, 0)),
            pl.BlockSpec((4, 64), lambda i: (0, 0)),
            pl.BlockSpec((4, 16), lambda i: (0, 0)),
        ],
        out_specs=[
            pl.BlockSpec((blk, 32), lambda i: (i, 0)),
            pl.BlockSpec((blk, 32), lambda i: (i, 0)),
            pl.BlockSpec((1000, 32), lambda i: (0, 0)),
            pl.BlockSpec((4, 32), lambda i: (0, 0)),
        ],
        out_shape=[
            jax.ShapeDtypeStruct((100000, 32), f32),
            jax.ShapeDtypeStruct((100000, 32), f32),
            jax.ShapeDtypeStruct((1000, 32), f32),
            jax.ShapeDtypeStruct((4, 32), f32),
        ],
    )(atc, ordt, cate, wti, wtc, prow4, ptyp4)


_MESH = dict(core_axis_name="c", subcore_axis_name="s")


def _zero16(ref, nrow):
    def b(r, _):
        ref[r, :] = jnp.zeros((16,), f32)
        return _
    lax.fori_loop(0, nrow, b, None)


def _edge_pass(g, P, S, CNT, cbuf, srcs, dsts, sv2d, dv2d, Grow, sem, wid,
               do_counts):
    pltpu.sync_copy(srcs.at[g, pl.ds(wid * ECH, ECH)], sv2d)
    pltpu.sync_copy(dsts.at[g, pl.ds(wid * ECH, ECH)], dv2d)

    def chunk(j, _):
        pltpu.async_copy(P.at[sv2d.at[j]], Grow, sem).wait()
        pltpu.sync_copy(Grow, S.at[dv2d.at[j]], add=True)
        if do_counts:
            def crow(i, _):
                s16 = sv2d[j, pl.ds(i * 16, 16)]
                d16 = dv2d[j, pl.ds(i * 16, 16)]
                cbuf[pl.ds(i * 16, 16)] = jnp.where(s16 == d16, 1.0, 0.0).astype(f32)
                return _
            lax.fori_loop(0, 8, crow, None)
            pltpu.sync_copy(cbuf, CNT.at[dv2d.at[j]], add=True)
        return _

    lax.fori_loop(0, ECH, chunk, None)


GRPT = NP // 32
GCH = GRPT // 64


def _sc_gather_body(atcT, ordT, cateT, item_ids, cate_ids, promoT,
                    xn_o, xr_o,
                    it1d, ct1d, Gi2, Gc2, bufN2, bufR2, ptT, sem, semW):
    cid = lax.axis_index("c")
    sid = lax.axis_index("s")
    wid = cid * 16 + sid
    rowbase = wid * GRPT
    pltpu.sync_copy(promoT, ptT)

    for g in range(4):
        tbl = atcT if g in (0, 2) else ordT
        pltpu.sync_copy(item_ids.at[pl.ds(g * NP + rowbase, GRPT)], it1d)
        pltpu.sync_copy(cate_ids.at[pl.ds(g * NP + rowbase, GRPT)], ct1d)

        def fire_gathers(c, slot):
            pltpu.async_copy(tbl.at[it1d.at[pl.ds(c * 64, 64)]],
                             Gi2.at[slot], sem)
            pltpu.async_copy(cateT.at[ct1d.at[pl.ds(c * 64, 64)]],
                             Gc2.at[slot], sem)

        def drain_gathers(c, slot):
            pltpu.make_async_copy(tbl.at[it1d.at[pl.ds(c * 64, 64)]],
                                  Gi2.at[slot], sem).wait()
            pltpu.make_async_copy(cateT.at[ct1d.at[pl.ds(c * 64, 64)]],
                                  Gc2.at[slot], sem).wait()

        def fire_writes(c, slot):
            base = rowbase + c * 64
            pltpu.async_copy(bufN2.at[slot], xn_o.at[g, pl.ds(base, 64)], semW)
            pltpu.async_copy(bufR2.at[slot], xr_o.at[g, pl.ds(base, 64)], semW)

        def drain_writes(c, slot):
            base = rowbase + c * 64
            pltpu.make_async_copy(bufN2.at[slot], xn_o.at[g, pl.ds(base, 64)],
                                  semW).wait()
            pltpu.make_async_copy(bufR2.at[slot], xr_o.at[g, pl.ds(base, 64)],
                                  semW).wait()

        fire_gathers(0, 0)

        def cbody(c, _):
            slot = lax.rem(c, 2)

            @pl.when(c >= 2)
            def _():
                drain_writes(c - 2, slot)
            drain_gathers(c, slot)

            @pl.when(c + 1 < GCH)
            def _():
                fire_gathers(c + 1, 1 - slot)

            def addrow(r, _):
                bufN2[slot, r, :] = Gi2[slot, r, 0:16] + Gc2[slot, r, 0:16]
                bufR2[slot, r, :] = (Gi2[slot, r, pl.ds(16, 16)]
                                     + Gc2[slot, r, pl.ds(16, 16)])
                return _
            lax.fori_loop(0, 64, addrow, None)

            @pl.when((wid == 0) & (c == 0))
            def _():
                bufN2[0, 0, :] = ptT[g, 0:16]
                bufR2[0, 0, :] = ptT[g, pl.ds(16, 16)]

            fire_writes(c, slot)
            return _

        lax.fori_loop(0, GCH, cbody, None)
        drain_writes(GCH - 2, GCH % 2)
        drain_writes(GCH - 1, (GCH - 1) % 2)


def _sc_gather_call(atcT, ordT, cateT, item1d, cate1d, promoT):
    kern = pl.kernel(
        _sc_gather_body,
        out_type=[
            jax.ShapeDtypeStruct((4, NP, 16), f32),
            jax.ShapeDtypeStruct((4, NP, 16), f32),
        ],
        mesh=plsc.VectorSubcoreMesh(**_MESH),
        compiler_params=pltpu.CompilerParams(use_tc_tiling_on_sc=False),
        scratch_types=[
            pltpu.VMEM((GRPT,), i32),
            pltpu.VMEM((GRPT,), i32),
            pltpu.VMEM((2, 64, 32), f32),
            pltpu.VMEM((2, 64, 32), f32),
            pltpu.VMEM((2, 64, 16), f32),
            pltpu.VMEM((2, 64, 16), f32),
            pltpu.VMEM((4, 32), f32),
            pltpu.SemaphoreType.DMA,
            pltpu.SemaphoreType.DMA,
        ],
    )
    return kern(atcT, ordT, cateT, item1d, cate1d, promoT)


def _make_sc_edge_body(do_counts):
    G = 2
    NGRP = ECH // G

    def body(*args):
        if do_counts:
            (tblH, srcs, dsts, Sp_o, cnt_o,
             bufN, sv4, dv4, Grow2, cbuf2, zb, zc, P, S, CNT,
             sem, semS, semI) = args
        else:
            (tblH, srcs, dsts, Sp_o,
             bufN, sv4, dv4, Grow2, cbuf2, zb, zc, P, S, CNT,
             sem, semS, semI) = args
            cnt_o = None
        cid = lax.axis_index("c")
        sid = lax.axis_index("s")
        wid = cid * 16 + sid
        rowbase = sid * RPT
        _zero16(zb, 128)

        def zc_b(i, _):
            zc[pl.ds(i * 16, 16)] = jnp.zeros((16,), f32)
            return _
        lax.fori_loop(0, 8, zc_b, None)

        GB = G * 128

        for g in range(4):
            tblg0 = tblH.at[g]
            for c in range(NCH):
                base = rowbase + c * 128
                pltpu.async_copy(tblg0.at[pl.ds(base, 128)],
                                 P.at[pl.ds(base, 128)], semI)
                pltpu.async_copy(zb, S.at[pl.ds(base, 128)], sem)
                if do_counts:
                    pltpu.async_copy(zc, CNT.at[pl.ds(base, 128)], sem)
            for c in range(NCH):
                base = rowbase + c * 128
                pltpu.make_async_copy(tblg0.at[pl.ds(base, 128)],
                                      P.at[pl.ds(base, 128)], semI).wait()
                pltpu.make_async_copy(zb, S.at[pl.ds(base, 128)], sem).wait()
                if do_counts:
                    pltpu.make_async_copy(zc, CNT.at[pl.ds(base, 128)], sem).wait()
            plsc.subcore_barrier()

            tblg = P
            srcf = srcs.at[g]
            dstf = dsts.at[g]

            def fire_idx(grp):
                s4 = lax.rem(grp, 4)
                row0 = wid * ECH + grp * G
                pltpu.async_copy(srcf.at[pl.ds(row0, G)],
                                 sv4.at[s4], semI)
                pltpu.async_copy(dstf.at[pl.ds(row0, G)],
                                 dv4.at[s4], semI)

            def drain_idx(grp):
                s4 = lax.rem(grp, 4)
                row0 = wid * ECH + grp * G
                pltpu.make_async_copy(srcf.at[pl.ds(row0, G)],
                                      sv4.at[s4], semI).wait()
                pltpu.make_async_copy(dstf.at[pl.ds(row0, G)],
                                      dv4.at[s4], semI).wait()

            def fire_gathers(grp, s2):
                s4 = lax.rem(grp, 4)
                for k in range(G):
                    pltpu.async_copy(
                        tblg.at[sv4.at[s4, k]],
                        Grow2.at[s2, pl.ds(k * 128, 128)], sem)

            def drain_gathers(grp, s2):
                s4 = lax.rem(grp, 4)
                for k in range(G):
                    pltpu.make_async_copy(
                        tblg.at[sv4.at[s4, k]],
                        Grow2.at[s2, pl.ds(k * 128, 128)], sem).wait()

            def fire_scatters(grp, s2):
                s4 = lax.rem(grp, 4)
                for k in range(G):
                    pltpu.async_copy(
                        Grow2.at[s2, pl.ds(k * 128, 128)],
                        S.at[dv4.at[s4, k]], semS, add=True)
                    if do_counts:
                        def crow(i, _):
                            s16 = sv4[s4, k, pl.ds(i * 16, 16)]
                            d16 = dv4[s4, k, pl.ds(i * 16, 16)]
                            cbuf2[s2, pl.ds(k * 128 + i * 16, 16)] = jnp.where(
                                s16 == d16, 1.0, 0.0).astype(f32)
                            return _
                        lax.fori_loop(0, 8, crow, None)
                        pltpu.async_copy(
                            cbuf2.at[s2, pl.ds(k * 128, 128)],
                            CNT.at[dv4.at[s4, k]], semS, add=True)

            def drain_scatters(grp, s2):
                s4 = lax.rem(grp, 4)
                for k in range(G):
                    pltpu.make_async_copy(
                        Grow2.at[s2, pl.ds(k * 128, 128)],
                        S.at[dv4.at[s4, k]], semS).wait()
                    if do_counts:
                        pltpu.make_async_copy(
                            cbuf2.at[s2, pl.ds(k * 128, 128)],
                            CNT.at[dv4.at[s4, k]], semS).wait()

            fire_idx(0)
            fire_idx(1)
            drain_idx(0)
            fire_gathers(0, 0)

            def grp_body(grp, _):
                s3 = lax.rem(grp, 3)
                drain_gathers(grp, s3)
                fire_scatters(grp, s3)

                @pl.when(grp >= 2)
                def _():
                    drain_scatters(grp - 2, lax.rem(grp + 1, 3))

                @pl.when(grp + 2 < NGRP)
                def _():
                    fire_idx(grp + 2)

                @pl.when(grp + 1 < NGRP)
                def _():
                    drain_idx(grp + 1)
                    fire_gathers(grp + 1, lax.rem(grp + 1, 3))
                return _

            lax.fori_loop(0, NGRP, grp_body, None)
            drain_scatters(NGRP - 2, (NGRP - 2) % 3)
            drain_scatters(NGRP - 1, (NGRP - 1) % 3)
            plsc.subcore_barrier()

            def ochunk(c, _):
                base = rowbase + c * 128
                pltpu.sync_copy(S.at[pl.ds(base, 128)], bufN)
                pltpu.sync_copy(bufN, Sp_o.at[cid, g, pl.ds(base, 128)])
                if do_counts:
                    pltpu.sync_copy(CNT.at[pl.ds(base, 128)], cbuf2.at[0, pl.ds(0, 128)])
                    pltpu.sync_copy(
                        cbuf2.at[0, pl.ds(0, 128)],
                        cnt_o.at[pl.ds((cid * 4 + g) * NP + base, 128)])
                return _
            lax.fori_loop(0, NCH, ochunk, None)
    return body


def _sc_edge_call(tbl, srcs3, dsts3, do_counts):
    out_type = [jax.ShapeDtypeStruct((2, 4, NP, 16), f32)]
    if do_counts:
        out_type.append(jax.ShapeDtypeStruct((2 * 4 * NP,), f32))
    kern = pl.kernel(
        _make_sc_edge_body(do_counts),
        out_type=out_type,
        mesh=plsc.VectorSubcoreMesh(**_MESH),
        compiler_params=pltpu.CompilerParams(use_tc_tiling_on_sc=False),
        scratch_types=[
            pltpu.VMEM((128, 16), f32),
            pltpu.VMEM((4, 2, 128), i32),
            pltpu.VMEM((4, 2, 128), i32),
            pltpu.VMEM((3, 256, 16), f32),
            pltpu.VMEM((3, 256), f32),
            pltpu.VMEM((128, 16), f32),
            pltpu.VMEM((128,), f32),
            pltpu.VMEM_SHARED((NP, 16), f32),
            pltpu.VMEM_SHARED((NP, 16), f32),
            pltpu.VMEM_SHARED((NP,), f32),
            pltpu.SemaphoreType.DMA,
            pltpu.SemaphoreType.DMA,
            pltpu.SemaphoreType.DMA,
        ],
    )
    return kern(tbl, srcs3, dsts3)


def _fkey(x):
    k = lax.bitcast_convert_type(x, i32)
    return k ^ jnp.where(k < 0, jnp.int32(0x7FFFFFFF), jnp.int32(0))


def _kth_threshold(keyT, k):
    def b(i, T):
        cand = T + (jnp.int32(1) << (30 - i))
        cnt = jnp.sum((keyT >= cand).astype(i32))
        return jnp.where(cnt >= k, cand, T)
    return lax.fori_loop(0, 31, b, jnp.int32(-2**31))


def _fold8(v, op):
    parts = [lax.slice_in_dim(v, k * 16, (k + 1) * 16) for k in range(8)]
    return functools.reduce(op, parts)


def _select_gate_pool(h, score_raw, alive, kcount, E8):
    scorem = jnp.where(alive, score_raw, -jnp.inf)
    key = _fkey(scorem)
    keyT = _fkey(jnp.transpose(scorem))
    T = _kth_threshold(keyT, kcount)
    keep = key >= T
    keepf = keep.astype(f32)
    gate = jnp.tanh(jnp.where(alive, score_raw, 0.0)) * keepf
    gexp = jnp.dot(gate, E8, preferred_element_type=f32)
    kexp = jnp.dot(keepf, E8, preferred_element_type=f32)
    g = h * gexp
    colmax = jnp.max(jnp.where(kexp > 0.0, g, -jnp.inf), axis=0)
    colsum = jnp.sum(g, axis=0)
    m16 = _fold8(colmax, jnp.maximum)
    s16 = _fold8(colsum, jnp.add) * (1.0 / kcount)
    return g, keepf, jnp.concatenate([m16, s16])


NSL = 4
NFS = NF // NSL


def _tc_h1_body(xnf_ref, xrf_ref, S0p_ref, cnt_ref, E8_ref, b0t_ref, h1_ref):
    S0 = S0p_ref[0, 0] + S0p_ref[1, 0]
    c8 = cnt_ref[0, 0] + cnt_ref[1, 0]
    cexp = jnp.dot(c8, E8_ref[...], preferred_element_type=f32)
    h1_ref[0] = jnp.maximum(
        S0 + (1.0 - cexp) * xnf_ref[0] + xrf_ref[0] + b0t_ref[...], 0.0)


def _tc_h1_call(xnf, xrf, S0pf, cnt8, E8, b0t):
    return pl.pallas_call(
        _tc_h1_body,
        grid=(4, NSL),
        in_specs=[
            pl.BlockSpec((1, NFS, 128), lambda i, j: (i, j, 0)),
            pl.BlockSpec((1, NFS, 128), lambda i, j: (i, j, 0)),
            pl.BlockSpec((2, 1, NFS, 128), lambda i, j: (0, i, j, 0)),
            pl.BlockSpec((2, 1, NFS, 8), lambda i, j: (0, i, j, 0)),
            pl.BlockSpec((8, 128), lambda i, j: (0, 0)),
            pl.BlockSpec((1, 128), lambda i, j: (0, 0)),
        ],
        out_specs=[pl.BlockSpec((1, NFS, 128), lambda i, j: (i, j, 0))],
        out_shape=[jax.ShapeDtypeStruct((4, NF, 128), f32)],
    )(xnf, xrf, S0pf, cnt8, E8, b0t)


def _tc_b_body(h1_ref, E8_ref, P0n_ref, Wn1b_ref, Wr1b_ref,
               pre2_ref, root2_ref, keep_ref, pool1_ref):
    h1 = h1_ref[0]
    E8 = E8_ref[...]
    score = jnp.dot(h1, P0n_ref[...], preferred_element_type=f32)
    r_iota = lax.broadcasted_iota(i32, (NF, 8), 0)
    k_iota = lax.broadcasted_iota(i32, (NF, 8), 1)
    valid = (r_iota * 8 + k_iota) < N
    g1, keepf, pool = _select_gate_pool(h1, score, valid, K1, E8)
    pool1_ref[0, 0] = pool
    keep_ref[0] = keepf
    pre2_ref[0] = jnp.dot(g1, Wn1b_ref[...], preferred_element_type=f32)
    root2_ref[0] = jnp.dot(g1, Wr1b_ref[...], preferred_element_type=f32)


def _tc_b_call(h1f, E8, P0n, Wn1b, Wr1b):
    return pl.pallas_call(
        _tc_b_body,
        grid=(4,),
        in_specs=[
            pl.BlockSpec((1, NF, 128), lambda i: (i, 0, 0)),
            pl.BlockSpec((8, 128), lambda i: (0, 0)),
            pl.BlockSpec((128, 8), lambda i: (0, 0)),
            pl.BlockSpec((128, 128), lambda i: (0, 0)),
            pl.BlockSpec((128, 128), lambda i: (0, 0)),
        ],
        out_specs=[
            pl.BlockSpec((1, NF, 128), lambda i: (i, 0, 0)),
            pl.BlockSpec((1, NF, 128), lambda i: (i, 0, 0)),
            pl.BlockSpec((1, NF, 8), lambda i: (i, 0, 0)),
            pl.BlockSpec((1, 1, 32), lambda i: (i, 0, 0)),
        ],
        out_shape=[
            jax.ShapeDtypeStruct((4, NF, 128), f32),
            jax.ShapeDtypeStruct((4, NF, 128), f32),
            jax.ShapeDtypeStruct((4, NF, 8), f32),
            jax.ShapeDtypeStruct((4, 1, 32), f32),
        ],
    )(h1f, E8, P0n, Wn1b, Wr1b)


def _tc_h2_body(S1p_ref, pre2_ref, root2_ref, keep_ref, cnt_ref, E8_ref,
                b1t_ref, h2_ref):
    S1 = S1p_ref[0, 0] + S1p_ref[1, 0]
    c8 = cnt_ref[0, 0] + cnt_ref[1, 0]
    E8 = E8_ref[...]
    cexp = jnp.dot(c8, E8, preferred_element_type=f32)
    kexp = jnp.dot(keep_ref[0], E8, preferred_element_type=f32)
    h2_ref[0] = kexp * jnp.maximum(
        S1 + (1.0 - cexp) * pre2_ref[0] + root2_ref[0] + b1t_ref[...], 0.0)


def _tc_h2_call(S1pf, pre2f, root2f, keep8, cnt8, E8, b1t):
    return pl.pallas_call(
        _tc_h2_body,
        grid=(4, NSL),
        in_specs=[
            pl.BlockSpec((2, 1, NFS, 128), lambda i, j: (0, i, j, 0)),
            pl.BlockSpec((1, NFS, 128), lambda i, j: (i, j, 0)),
            pl.BlockSpec((1, NFS, 128), lambda i, j: (i, j, 0)),
            pl.BlockSpec((1, NFS, 8), lambda i, j: (i, j, 0)),
            pl.BlockSpec((2, 1, NFS, 8), lambda i, j: (0, i, j, 0)),
            pl.BlockSpec((8, 128), lambda i, j: (0, 0)),
            pl.BlockSpec((1, 128), lambda i, j: (0, 0)),
        ],
        out_specs=[pl.BlockSpec((1, NFS, 128), lambda i, j: (i, j, 0))],
        out_shape=[jax.ShapeDtypeStruct((4, NF, 128), f32)],
    )(S1pf, pre2f, root2f, keep8, cnt8, E8, b1t)


def _tc_d1_body(h2_ref, keep_ref, pool1_ref, E8_ref, P1n_ref, pooltot_ref):
    h2 = h2_ref[0]
    keep1 = keep_ref[0]
    E8 = E8_ref[...]
    score = jnp.dot(h2, P1n_ref[...], preferred_element_type=f32)
    alive = keep1 > 0.0
    _, _, pool = _select_gate_pool(h2, score, alive, K2, E8)
    pooltot_ref[0, 0] = pool1_ref[0, 0] + pool


def _tc_d1_call(h2f, keep8, pool1, E8, P1n):
    return pl.pallas_call(
        _tc_d1_body,
        grid=(4,),
        in_specs=[
            pl.BlockSpec((1, NF, 128), lambda i: (i, 0, 0)),
            pl.BlockSpec((1, NF, 8), lambda i: (i, 0, 0)),
            pl.BlockSpec((1, 1, 32), lambda i: (i, 0, 0)),
            pl.BlockSpec((8, 128), lambda i: (0, 0)),
            pl.BlockSpec((128, 8), lambda i: (0, 0)),
        ],
        out_specs=[pl.BlockSpec((1, 1, 32), lambda i: (i, 0, 0))],
        out_shape=[jax.ShapeDtypeStruct((4, 1, 32), f32)],
    )(h2f, keep8, pool1, E8, P1n)


def _tc_d2_body(pool_ref, wl_ref, bl_ref, prow_ref, sig_ref, fc_ref):
    xx = jnp.concatenate(
        [pool_ref[0], pool_ref[1], pool_ref[2], pool_ref[3]],
        axis=1)
    fc = jnp.dot(xx, wl_ref[...], preferred_element_type=f32) + bl_ref[...]
    pv = prow_ref[0:1, :]
    s = jnp.sum(pv * fc)
    sig_ref[...] = (1.0 / (1.0 + jnp.exp(-s)))[None]
    fc_ref[...] = fc[0]


def _tc_d2_call(pooltot, W_lin, b_lin, prows):
    return pl.pallas_call(
        _tc_d2_body,
        out_shape=[
            jax.ShapeDtypeStruct((1,), f32),
            jax.ShapeDtypeStruct((64,), f32),
        ],
    )(pooltot, W_lin, b_lin, prows)


def kernel(x_b_atc, edge_index_b_atc, x_b_ord, edge_index_b_ord,
           x_a_atc, edge_index_a_atc, x_a_ord, edge_index_a_ord,
           atc_emb, ord_emb, promo_emb, promo_type_emb, cate_emb,
           Wr0, Wn0, b0, Wr1, Wn1, b1, p0, p1, W_lin, b_lin):
    xs = [x_b_atc, x_b_ord, x_a_atc, x_a_ord]
    eis = [edge_index_b_atc, edge_index_b_ord, edge_index_a_atc,
           edge_index_a_ord]

    npad = NP - N
    item3 = jnp.stack([
        jnp.pad(x[:, 0] % 100000, (0, npad)) for x in xs]).reshape(4 * NP)
    cate3 = jnp.stack([
        jnp.pad(x[:, 1] % 1000, (0, npad)) for x in xs]).reshape(4 * NP)
    epad = EP - E
    srcs3 = jnp.stack([
        jnp.pad(ei[0], (0, epad)) for ei in eis]).reshape(4, EP // 128, 128)
    padvals = N + (jnp.arange(epad, dtype=jnp.int32) % (NP - N))
    dsts3 = jnp.stack([
        jnp.concatenate([ei[1], padvals]) for ei in eis
    ]).reshape(4, EP // 128, 128)
    pids4 = jnp.stack([x[0, 0] % 100000 for x in xs])
    ptypes4 = jnp.stack([x[0, 1] % 100 for x in xs])
    prow4 = jnp.take(promo_emb, pids4, axis=0)
    ptyp4 = jnp.take(promo_type_emb, ptypes4, axis=0)

    wti = jnp.concatenate([Wn0[:64], Wr0[:64]], axis=1)
    wtc = jnp.concatenate([Wn0[64:], Wr0[64:]], axis=1)
    eye8 = jnp.eye(8, dtype=f32)
    E8 = jnp.kron(eye8, jnp.ones((1, 16), f32))
    P0n = jnp.kron(eye8, (p0 / jnp.linalg.norm(p0))[:, None])
    P1n = jnp.kron(eye8, (p1 / jnp.linalg.norm(p1))[:, None])
    Wn1b = jnp.kron(eye8, Wn1)
    Wr1b = jnp.kron(eye8, Wr1)
    b0t = jnp.tile(b0, 8)[None, :]
    b1t = jnp.tile(b1, 8)[None, :]

    atcT, ordT, cateT, promoT = _prep_call(
        atc_emb, ord_emb, cate_emb, wti, wtc, prow4, ptyp4)

    xn, xr = _sc_gather_call(atcT, ordT, cateT, item3, cate3, promoT)
    S0p, cnt = _sc_edge_call(xn, srcs3, dsts3, do_counts=True)

    xnf = xn.reshape(4, NF, 128)
    xrf = xr.reshape(4, NF, 128)
    S0pf = S0p.reshape(2, 4, NF, 128)
    cnt8 = cnt.reshape(2, 4, NF, 8)

    h1f = _tc_h1_call(xnf, xrf, S0pf, cnt8, E8, b0t)[0]
    pre2f, root2f, keep8, pool1 = _tc_b_call(h1f, E8, P0n, Wn1b, Wr1b)

    S1p = _sc_edge_call(pre2f.reshape(4, NP, 16), srcs3, dsts3,
                        do_counts=False)[0]
    S1pf = S1p.reshape(2, 4, NF, 128)

    h2f = _tc_h2_call(S1pf, pre2f, root2f, keep8, cnt8, E8, b1t)[0]
    pooltot = _tc_d1_call(h2f, keep8, pool1, E8, P1n)[0]

    sig, fc = _tc_d2_call(pooltot, W_lin, b_lin, prow4)

    return (sig, xs[3][0, 0], prow4[0], xs[3][0, 1], fc)
